# baseline (speedup 1.0000x reference)
import functools
import jax
import jax.numpy as jnp
from jax import lax
from jax.experimental import pallas as pl
from jax.experimental.pallas import tpu as pltpu
from jax.experimental.pallas import tpu_sc as plsc

H, W = 256, 256
B, S = 4, 48
SEG = 2048
SIDX = 4
NSEG = 34
NUNIT3 = B * NSEG
NW = 32

_KVALS = [0.03125, 0.03125, 0.01562, 0.03125, 0.03125,
          0.03125, 0.04406, 0.03125, 0.04406, 0.03125,
          0.0425, 0.0625, 0.14502, 0.0625, 0.0425,
          0.03125, 0.04406, 0.03125, 0.04406, 0.03125,
          0.03125, 0.03125, 0.01562, 0.03125, 0.03125]
def _round_bf16(v):
    import struct
    bits = struct.unpack("<I", struct.pack("<f", float(v)))[0]
    lsb = (bits >> 16) & 1
    bits = (bits + 0x7FFF + lsb) & 0xFFFF0000
    return struct.unpack("<f", struct.pack("<I", bits))[0]


_KVALS_BF16 = [_round_bf16(v) for v in _KVALS]

@functools.lru_cache(maxsize=1)
def _sc_mesh():
    return plsc.VectorSubcoreMesh(core_axis_name="c", subcore_axis_name="s",
                                  num_cores=2, num_subcores=16)


def _k1_body(xy_hbm, out_hbm, ev_v, buf_v, red_v, sem):
    c = lax.axis_index("c")
    s = lax.axis_index("s")
    w = s * 2 + c
    zeros16 = jnp.zeros((16,), jnp.int32)
    ones16 = jnp.ones((16,), jnp.int32)
    lanebase = jnp.arange(16, dtype=jnp.int32) * 256
    NU = (2 * B * S) // NW

    def zero(j, _):
        for jj in range(8):
            buf_v[pl.ds(j * 128 + jj * 16, 16)] = zeros16
        return 0
    lax.fori_loop(0, 32, zero, 0)

    def ev_dma(i, ib):
        u = w + NW * i
        return pltpu.async_copy(
            xy_hbm.at[u // S, pl.ds((u % S) * SEG, SEG)], ev_v.at[ib], sem)

    pend = ev_dma(0, 0)
    for i in range(NU):
        u = w + NW * i
        nxt = ev_dma(i + 1, (i + 1) % 2) if i + 1 < NU else None
        pend.wait()
        pend = nxt
        ib = i % 2

        def scat(j, _):
            for jj in range(8):
                v = ev_v[ib, pl.ds(j * 128 + jj * 16, 16)]
                plsc.addupdate_scatter(buf_v, [v + lanebase], ones16)
            return 0
        lax.fori_loop(0, SEG // 128, scat, 0)

        def red(jc, _):
            acc = zeros16
            for l in range(16):
                off = l * 256 + jc * 16
                acc = acc + buf_v[pl.ds(off, 16)]
                buf_v[pl.ds(off, 16)] = zeros16
            red_v[pl.ds(jc * 16, 16)] = acc
            return 0
        lax.fori_loop(0, 16, red, 0)
        pltpu.sync_copy(red_v, out_hbm.at[u // S, u % S])


@functools.lru_cache(maxsize=1)
def _k1():
    return pl.kernel(
        _k1_body,
        out_type=jax.ShapeDtypeStruct((2 * B, S, 256), jnp.int32),
        mesh=_sc_mesh(),
        compiler_params=pltpu.CompilerParams(needs_layout_passes=False),
        scratch_types=[
            pltpu.VMEM((2, SEG), jnp.int32),
            pltpu.VMEM((4096,), jnp.int32),
            pltpu.VMEM((256,), jnp.int32),
            pltpu.SemaphoreType.DMA,
        ],
    )


def _sortnet(rows):
    rows = list(rows)
    n = len(rows)
    for _ in range(n):
        for p0 in (0, 1):
            for i in range(p0, n - 1, 2):
                lo = jnp.minimum(rows[i], rows[i + 1])
                hi = jnp.maximum(rows[i], rows[i + 1])
                rows[i], rows[i + 1] = lo, hi
    return rows


def _skip_from_mean(mean_col):
    rows = [mean_col[5 + j:38 + j] for j in range(10)]
    srt = _sortnet(rows)
    med = (srt[4] + srt[5]) * 0.5
    diffs = [jnp.sqrt((r - med) ** 2) for r in rows]
    d0 = diffs[0]
    dsrt = _sortnet(diffs)
    mad = (dsrt[4] + dsrt[5]) * 0.5
    mad = jnp.where(mad == 0, jnp.float32(1e-12), mad)
    return (0.6745 * d0 / mad) > 3.0


def k2_body(ax_ref, ay_ref, alx_ref, aly_ref, skip_ref):
    wv = lax.broadcasted_iota(jnp.int32, (1, 256), 1).astype(jnp.float32)

    def mean_col(a):
        a = a.astype(jnp.bfloat16).astype(jnp.float32)
        z2r = jnp.zeros((2, 256), jnp.float32)
        p = jnp.concatenate([z2r, a, z2r], axis=0)
        z2c = jnp.zeros((S + 4, 2), jnp.float32)
        p = jnp.concatenate([z2c, p, z2c], axis=1)
        acc = jnp.zeros((S, 256), jnp.float32)
        for ds in range(5):
            for dw in range(5):
                acc = acc + _KVALS_BF16[ds * 5 + dw] * p[ds:ds + S, dw:dw + 256]
        acc = acc.astype(jnp.bfloat16).astype(jnp.float32)
        return jnp.sum(acc * wv, axis=1, keepdims=True) / float(SEG)

    mx = mean_col(ax_ref[0].astype(jnp.float32))
    my = mean_col(ay_ref[0].astype(jnp.float32))
    alx_ref[0] = jnp.round(mx - (W // 2)).astype(jnp.int32)
    aly_ref[0] = jnp.round(my - (H // 2)).astype(jnp.int32)
    sk = _skip_from_mean(mx) | _skip_from_mean(my)
    skip_ref[0] = sk.astype(jnp.int32)


_k2 = pl.pallas_call(
    k2_body,
    grid=(B,),
    in_specs=[
        pl.BlockSpec((1, S, 256), lambda b: (b, 0, 0)),
        pl.BlockSpec((1, S, 256), lambda b: (b, 0, 0)),
    ],
    out_specs=[
        pl.BlockSpec((1, S, 1), lambda b: (b, 0, 0)),
        pl.BlockSpec((1, S, 1), lambda b: (b, 0, 0)),
        pl.BlockSpec((1, 33, 1), lambda b: (b, 0, 0)),
    ],
    out_shape=[
        jax.ShapeDtypeStruct((B, S, 1), jnp.int32),
        jax.ShapeDtypeStruct((B, S, 1), jnp.int32),
        jax.ShapeDtypeStruct((B, 33, 1), jnp.int32),
    ],
)


def _k3_body(bsel, xy_hbm, ax_hbm, ay_hbm, out_hbm,
             xv, yv, idxv, onesv, zerov, zv, axv, ayv, shared,
             evsem, addsem, outsem):
    c = lax.axis_index("c")
    s = lax.axis_index("s")
    w = s * 2 + c
    b = bsel
    pltpu.sync_copy(ax_hbm, axv)
    pltpu.sync_copy(ay_hbm, ayv)
    ones16 = jnp.ones((16,), jnp.int32)
    zeros16 = jnp.zeros((16,), jnp.int32)
    for j in range(8):
        onesv[pl.ds(j * 16, 16)] = ones16
        zerov[pl.ds(j * 16, 16)] = zeros16

    def zfill(j, _):
        zv[pl.ds(j * 16, 16)] = zeros16
        return 0
    lax.fori_loop(0, 512, zfill, 0)

    base = s * (W * H)

    def zspm(j, _):
        pltpu.sync_copy(zv, shared.at[pl.ds(base + j * 8192, 8192)])
        return 0
    lax.fori_loop(0, (W * H) // 8192, zspm, 0)

    NU = (NSEG + NW - 1) // NW

    def load_events(i):
        u = w + NW * i
        si = u % NSEG + SIDX
        d1 = pltpu.async_copy(xy_hbm.at[b, pl.ds(si * SEG, SEG)], xv, evsem)
        d2 = pltpu.async_copy(xy_hbm.at[b + B, pl.ds(si * SEG, SEG)], yv, evsem)
        d1.wait()
        d2.wait()

    def compute_idx(i, ib):
        u = w + NW * i
        si = u % NSEG + SIDX
        axs = plsc.load_gather(axv, [jnp.full((16,), b * S + si, jnp.int32)])
        ays = plsc.load_gather(ayv, [jnp.full((16,), b * S + si, jnp.int32)])

        def row(r, _):
            for cc in range(8):
                off = r * 128 + cc * 16
                xi = xv[pl.ds(off, 16)]
                yi = yv[pl.ds(off, 16)]
                xi = jnp.minimum(jnp.maximum(xi - axs, 0), W - 1)
                yi = jnp.minimum(jnp.maximum(yi - ays, 0), H - 1)
                idxv[ib, r, pl.ds(cc * 16, 16)] = base + xi + W * yi
            return 0
        lax.fori_loop(0, 16, row, 0)

    @pl.when(w < NSEG)
    def _():
        load_events(0)
        compute_idx(0, 0)

    cur = 0
    for i in range(NU):
        u = w + NW * i

        @pl.when(u < NSEG)
        def _(i=i, cur=cur, u=u):
            adds = [pltpu.async_copy(onesv, shared.at[idxv.at[cur, r]],
                                     addsem, add=True) for r in range(16)]
            for d in adds:
                d.wait()
            dout = pltpu.async_copy(shared.at[pl.ds(base, W * H)],
                                    out_hbm.at[u], outsem)

            @pl.when(u + NW < NSEG)
            def _():
                load_events(i + 1)
                compute_idx(i + 1, 1 - cur)
            dout.wait()

            @pl.when(u + NW < NSEG)
            def _():
                zs = [pltpu.async_copy(zerov, shared.at[idxv.at[cur, r]],
                                       addsem) for r in range(16)]
                for d in zs:
                    d.wait()
        cur = 1 - cur


@functools.lru_cache(maxsize=4)
def _k3(bsel):
    return pl.kernel(
        functools.partial(_k3_body, bsel),
        out_type=jax.ShapeDtypeStruct((NSEG, W * H), jnp.int32),
        mesh=_sc_mesh(),
        compiler_params=pltpu.CompilerParams(needs_layout_passes=False),
        scratch_types=[
            pltpu.VMEM((SEG,), jnp.int32),
            pltpu.VMEM((SEG,), jnp.int32),
            pltpu.VMEM((2, 16, 128), jnp.int32),
            pltpu.VMEM((128,), jnp.int32),
            pltpu.VMEM((128,), jnp.int32),
            pltpu.VMEM((8192,), jnp.int32),
            pltpu.VMEM((B * S,), jnp.int32),
            pltpu.VMEM((B * S,), jnp.int32),
            pltpu.VMEM_SHARED((16 * W * H,), jnp.int32),
            pltpu.SemaphoreType.DMA,
            pltpu.SemaphoreType.DMA,
            pltpu.SemaphoreType.DMA,
        ],
    )


def k4_body(skip_ref, hist_ref, out_ref):
    bf = jnp.bfloat16
    b = pl.program_id(0)
    h0 = hist_ref[0, 0]
    container = h0.astype(jnp.float32)
    v0 = (h0 > 0).astype(bf)
    v1 = jnp.ones_like(v0)
    confident = jnp.zeros_like(v0)
    active = jnp.float32(1.0)
    s0 = jnp.sum(v0.astype(jnp.float32))
    for k in range(1, NSEG):
        h = hist_ref[0, k]
        skipf = skip_ref[b, k - 1].astype(jnp.float32)
        proc = active * (1.0 - skipf)
        vn = (h > 0).astype(bf)
        confident = confident + proc.astype(bf) * (vn * v0 * v1)
        u = jnp.maximum(vn, v0)
        su = jnp.sum(u.astype(jnp.float32))
        brk = jnp.where((su - s0) / su < 0.01, 1.0, 0.0)
        cont = proc * (1.0 - brk)
        container = container + cont * h.astype(jnp.float32)
        cb = cont.astype(bf)
        v1 = v1 + cb * (v0 - v1)
        v0 = v0 + cb * (u - v0)
        s0 = s0 + cont * (su - s0)
        active = active * (1.0 - proc * brk)
    n = float(W * H)
    for plane, img in ((0, container), (1, confident.astype(jnp.float32))):
        m = jnp.sum(img) / n
        sd = jnp.sqrt(jnp.sum((img - m) ** 2) / (n - 1.0))
        cv = m + 3.0 * sd
        out_ref[0, plane] = jnp.clip(img, 0.0, cv) / cv


_k4 = pl.pallas_call(
    k4_body,
    grid=(1,),
    in_specs=[
        pl.BlockSpec(memory_space=pltpu.SMEM),
        pl.BlockSpec((1, NSEG, 256, 256), lambda b: (b, 0, 0, 0)),
    ],
    out_specs=pl.BlockSpec((1, 2, 256, 256), lambda b: (b, 0, 0, 0)),
    out_shape=jax.ShapeDtypeStruct((1, 2, H, W), jnp.float32),
    compiler_params=pltpu.CompilerParams(
        dimension_semantics=("arbitrary",)),
)


def kernel(events):
    n = events.shape[1]
    xy = events[..., :2].astype(jnp.int32).transpose(2, 0, 1).reshape(2 * B, n)
    along = _k1()(xy)
    alx, aly, skip = _k2(along[:B], along[B:])
    axf = alx.reshape(-1)
    ayf = aly.reshape(-1)
    skip2 = skip.reshape(B, 33)
    outs = []
    for b in range(B):
        hist = _k3(b)(xy, axf, ayf)
        outs.append(_k4(skip2[b:b + 1], hist.reshape(1, NSEG, 256, 256)))
    return jnp.concatenate(outs, axis=0)


# speedup vs baseline: 1.0910x; 1.0910x over previous
import functools
import jax
import jax.numpy as jnp
from jax import lax
from jax.experimental import pallas as pl
from jax.experimental.pallas import tpu as pltpu
from jax.experimental.pallas import tpu_sc as plsc

H, W = 256, 256
B, S = 4, 48
SEG = 2048
SIDX = 4
NSEG = 34
NUNIT3 = B * NSEG
NW = 32

_KVALS = [0.03125, 0.03125, 0.01562, 0.03125, 0.03125,
          0.03125, 0.04406, 0.03125, 0.04406, 0.03125,
          0.0425, 0.0625, 0.14502, 0.0625, 0.0425,
          0.03125, 0.04406, 0.03125, 0.04406, 0.03125,
          0.03125, 0.03125, 0.01562, 0.03125, 0.03125]
def _round_bf16(v):
    import struct
    bits = struct.unpack("<I", struct.pack("<f", float(v)))[0]
    lsb = (bits >> 16) & 1
    bits = (bits + 0x7FFF + lsb) & 0xFFFF0000
    return struct.unpack("<f", struct.pack("<I", bits))[0]


_KVALS_BF16 = [_round_bf16(v) for v in _KVALS]

@functools.lru_cache(maxsize=1)
def _sc_mesh():
    return plsc.VectorSubcoreMesh(core_axis_name="c", subcore_axis_name="s",
                                  num_cores=2, num_subcores=16)


def _k1_body(xy_hbm, out_hbm, ev_v, buf_v, red_v, sem):
    c = lax.axis_index("c")
    s = lax.axis_index("s")
    w = s * 2 + c
    zeros16 = jnp.zeros((16,), jnp.int32)
    ones16 = jnp.ones((16,), jnp.int32)
    lanebase = jnp.arange(16, dtype=jnp.int32) * 256
    NU = (2 * B * S) // NW

    def zero(j, _):
        for jj in range(8):
            buf_v[pl.ds(j * 128 + jj * 16, 16)] = zeros16
        return 0
    lax.fori_loop(0, 32, zero, 0)

    def ev_dma(i, ib):
        u = w + NW * i
        return pltpu.async_copy(
            xy_hbm.at[u // S, pl.ds((u % S) * SEG, SEG)], ev_v.at[ib], sem)

    pend = ev_dma(0, 0)
    for i in range(NU):
        u = w + NW * i
        nxt = ev_dma(i + 1, (i + 1) % 2) if i + 1 < NU else None
        pend.wait()
        pend = nxt
        ib = i % 2

        def scat(j, _):
            for jj in range(8):
                v = ev_v[ib, pl.ds(j * 128 + jj * 16, 16)]
                plsc.addupdate_scatter(buf_v, [v + lanebase], ones16)
            return 0
        lax.fori_loop(0, SEG // 128, scat, 0)

        def red(jc, _):
            acc = zeros16
            for l in range(16):
                off = l * 256 + jc * 16
                acc = acc + buf_v[pl.ds(off, 16)]
                buf_v[pl.ds(off, 16)] = zeros16
            red_v[pl.ds(jc * 16, 16)] = acc
            return 0
        lax.fori_loop(0, 16, red, 0)
        pltpu.sync_copy(red_v, out_hbm.at[u // S, u % S])


@functools.lru_cache(maxsize=1)
def _k1():
    return pl.kernel(
        _k1_body,
        out_type=jax.ShapeDtypeStruct((2 * B, S, 256), jnp.int32),
        mesh=_sc_mesh(),
        compiler_params=pltpu.CompilerParams(needs_layout_passes=False),
        scratch_types=[
            pltpu.VMEM((2, SEG), jnp.int32),
            pltpu.VMEM((4096,), jnp.int32),
            pltpu.VMEM((256,), jnp.int32),
            pltpu.SemaphoreType.DMA,
        ],
    )


def _sortnet(rows):
    rows = list(rows)
    n = len(rows)
    for _ in range(n):
        for p0 in (0, 1):
            for i in range(p0, n - 1, 2):
                lo = jnp.minimum(rows[i], rows[i + 1])
                hi = jnp.maximum(rows[i], rows[i + 1])
                rows[i], rows[i + 1] = lo, hi
    return rows


def _skip_from_mean(mean_col):
    rows = [mean_col[5 + j:38 + j] for j in range(10)]
    srt = _sortnet(rows)
    med = (srt[4] + srt[5]) * 0.5
    diffs = [jnp.sqrt((r - med) ** 2) for r in rows]
    d0 = diffs[0]
    dsrt = _sortnet(diffs)
    mad = (dsrt[4] + dsrt[5]) * 0.5
    mad = jnp.where(mad == 0, jnp.float32(1e-12), mad)
    return (0.6745 * d0 / mad) > 3.0


def k2_body(ax_ref, ay_ref, alx_ref, aly_ref, skip_ref):
    wv = lax.broadcasted_iota(jnp.int32, (1, 256), 1).astype(jnp.float32)

    def mean_col(a):
        a = a.astype(jnp.bfloat16).astype(jnp.float32)
        z2r = jnp.zeros((2, 256), jnp.float32)
        p = jnp.concatenate([z2r, a, z2r], axis=0)
        z2c = jnp.zeros((S + 4, 2), jnp.float32)
        p = jnp.concatenate([z2c, p, z2c], axis=1)
        acc = jnp.zeros((S, 256), jnp.float32)
        for ds in range(5):
            for dw in range(5):
                acc = acc + _KVALS_BF16[ds * 5 + dw] * p[ds:ds + S, dw:dw + 256]
        acc = acc.astype(jnp.bfloat16).astype(jnp.float32)
        return jnp.sum(acc * wv, axis=1, keepdims=True) / float(SEG)

    mx = mean_col(ax_ref[0].astype(jnp.float32))
    my = mean_col(ay_ref[0].astype(jnp.float32))
    alx_ref[0] = jnp.round(mx - (W // 2)).astype(jnp.int32)
    aly_ref[0] = jnp.round(my - (H // 2)).astype(jnp.int32)
    sk = _skip_from_mean(mx) | _skip_from_mean(my)
    skip_ref[0] = sk.astype(jnp.int32)


_k2 = pl.pallas_call(
    k2_body,
    grid=(B,),
    in_specs=[
        pl.BlockSpec((1, S, 256), lambda b: (b, 0, 0)),
        pl.BlockSpec((1, S, 256), lambda b: (b, 0, 0)),
    ],
    out_specs=[
        pl.BlockSpec((1, S, 1), lambda b: (b, 0, 0)),
        pl.BlockSpec((1, S, 1), lambda b: (b, 0, 0)),
        pl.BlockSpec((1, 33, 1), lambda b: (b, 0, 0)),
    ],
    out_shape=[
        jax.ShapeDtypeStruct((B, S, 1), jnp.int32),
        jax.ShapeDtypeStruct((B, S, 1), jnp.int32),
        jax.ShapeDtypeStruct((B, 33, 1), jnp.int32),
    ],
)


def _k3_body(xy_hbm, ax_hbm, ay_hbm, out_hbm,
             xv, yv, idxv, onesv, zerov, zv, axv, ayv, shared,
             evsem, addsem, outsem):
    c = lax.axis_index("c")
    s = lax.axis_index("s")
    w = s * 2 + c
    pltpu.sync_copy(ax_hbm, axv)
    pltpu.sync_copy(ay_hbm, ayv)
    ones16 = jnp.ones((16,), jnp.int32)
    zeros16 = jnp.zeros((16,), jnp.int32)
    for j in range(8):
        onesv[pl.ds(j * 16, 16)] = ones16
        zerov[pl.ds(j * 16, 16)] = zeros16

    def zfill(j, _):
        zv[pl.ds(j * 16, 16)] = zeros16
        return 0
    lax.fori_loop(0, 512, zfill, 0)

    base = s * (W * H)

    def zspm(j, _):
        pltpu.sync_copy(zv, shared.at[pl.ds(base + j * 8192, 8192)])
        return 0
    lax.fori_loop(0, (W * H) // 8192, zspm, 0)

    NU = (NUNIT3 + NW - 1) // NW

    def load_events(i):
        u = w + NW * i
        b = u // NSEG
        si = u % NSEG + SIDX
        d1 = pltpu.async_copy(xy_hbm.at[b, pl.ds(si * SEG, SEG)], xv, evsem)
        d2 = pltpu.async_copy(xy_hbm.at[b + B, pl.ds(si * SEG, SEG)], yv, evsem)
        d1.wait()
        d2.wait()

    def compute_idx(i, ib):
        u = w + NW * i
        b = u // NSEG
        si = u % NSEG + SIDX
        axs = plsc.load_gather(axv, [jnp.full((16,), b * S + si, jnp.int32)])
        ays = plsc.load_gather(ayv, [jnp.full((16,), b * S + si, jnp.int32)])

        def row(r, _):
            for cc in range(8):
                off = r * 128 + cc * 16
                xi = xv[pl.ds(off, 16)]
                yi = yv[pl.ds(off, 16)]
                xi = jnp.minimum(jnp.maximum(xi - axs, 0), W - 1)
                yi = jnp.minimum(jnp.maximum(yi - ays, 0), H - 1)
                idxv[ib, r, pl.ds(cc * 16, 16)] = base + xi + W * yi
            return 0
        lax.fori_loop(0, 16, row, 0)

    @pl.when(w < NUNIT3)
    def _():
        load_events(0)
        compute_idx(0, 0)

    cur = 0
    for i in range(NU):
        u = w + NW * i

        @pl.when(u < NUNIT3)
        def _(i=i, cur=cur, u=u):
            adds = [pltpu.async_copy(onesv, shared.at[idxv.at[cur, r]],
                                     addsem, add=True) for r in range(16)]
            for d in adds:
                d.wait()
            dout = pltpu.async_copy(shared.at[pl.ds(base, W * H)],
                                    out_hbm.at[u], outsem)

            @pl.when(u + NW < NUNIT3)
            def _():
                load_events(i + 1)
                compute_idx(i + 1, 1 - cur)
            dout.wait()

            @pl.when(u + NW < NUNIT3)
            def _():
                zs = [pltpu.async_copy(zerov, shared.at[idxv.at[cur, r]],
                                       addsem) for r in range(16)]
                for d in zs:
                    d.wait()
        cur = 1 - cur


@functools.lru_cache(maxsize=1)
def _k3():
    return pl.kernel(
        _k3_body,
        out_type=jax.ShapeDtypeStruct((NUNIT3, W * H), jnp.int32),
        mesh=_sc_mesh(),
        compiler_params=pltpu.CompilerParams(needs_layout_passes=False),
        scratch_types=[
            pltpu.VMEM((SEG,), jnp.int32),
            pltpu.VMEM((SEG,), jnp.int32),
            pltpu.VMEM((2, 16, 128), jnp.int32),
            pltpu.VMEM((128,), jnp.int32),
            pltpu.VMEM((128,), jnp.int32),
            pltpu.VMEM((8192,), jnp.int32),
            pltpu.VMEM((B * S,), jnp.int32),
            pltpu.VMEM((B * S,), jnp.int32),
            pltpu.VMEM_SHARED((16 * W * H,), jnp.int32),
            pltpu.SemaphoreType.DMA,
            pltpu.SemaphoreType.DMA,
            pltpu.SemaphoreType.DMA,
        ],
    )


def k4_body(skip_ref, hist_ref, out_ref):
    bf = jnp.bfloat16
    b = pl.program_id(0)
    h0 = hist_ref[0, 0]
    container = h0.astype(jnp.float32)
    v0 = (h0 > 0).astype(bf)
    v1 = jnp.ones_like(v0)
    confident = jnp.zeros_like(v0)
    active = jnp.float32(1.0)
    s0 = jnp.sum(v0.astype(jnp.float32))
    for k in range(1, NSEG):
        h = hist_ref[0, k]
        skipf = skip_ref[b, k - 1].astype(jnp.float32)
        proc = active * (1.0 - skipf)
        vn = (h > 0).astype(bf)
        confident = confident + proc.astype(bf) * (vn * v0 * v1)
        u = jnp.maximum(vn, v0)
        su = jnp.sum(u.astype(jnp.float32))
        brk = jnp.where((su - s0) / su < 0.01, 1.0, 0.0)
        cont = proc * (1.0 - brk)
        container = container + cont * h.astype(jnp.float32)
        cb = cont.astype(bf)
        v1 = v1 + cb * (v0 - v1)
        v0 = v0 + cb * (u - v0)
        s0 = s0 + cont * (su - s0)
        active = active * (1.0 - proc * brk)
    n = float(W * H)
    for plane, img in ((0, container), (1, confident.astype(jnp.float32))):
        m = jnp.sum(img) / n
        sd = jnp.sqrt(jnp.sum((img - m) ** 2) / (n - 1.0))
        cv = m + 3.0 * sd
        out_ref[0, plane] = jnp.clip(img, 0.0, cv) / cv


_k4 = pl.pallas_call(
    k4_body,
    grid=(B,),
    in_specs=[
        pl.BlockSpec(memory_space=pltpu.SMEM),
        pl.BlockSpec((1, NSEG, 256, 256), lambda b: (b, 0, 0, 0)),
    ],
    out_specs=pl.BlockSpec((1, 2, 256, 256), lambda b: (b, 0, 0, 0)),
    out_shape=jax.ShapeDtypeStruct((B, 2, H, W), jnp.float32),
    compiler_params=pltpu.CompilerParams(
        dimension_semantics=("arbitrary",)),
)


def kernel(events):
    n = events.shape[1]
    xy = events[..., :2].astype(jnp.int32).transpose(2, 0, 1).reshape(2 * B, n)
    along = _k1()(xy)
    alx, aly, skip = _k2(along[:B], along[B:])
    hist = _k3()(xy, alx.reshape(-1), aly.reshape(-1))
    out = _k4(skip.reshape(B, 33), hist.reshape(B, NSEG, 256, 256))
    return out
